# baseline (speedup 1.0000x reference)
import functools

import jax
import jax.numpy as jnp
from jax import lax
from jax.experimental import pallas as pl
from jax.experimental.pallas import tpu as pltpu
from jax.experimental.pallas import tpu_sc as plsc

_NUM_CORES = 2
_LANES = 16


def kernel(features, query_letters):
    n, h = features.shape
    mesh = plsc.VectorSubcoreMesh(core_axis_name="c", subcore_axis_name="s")

    @functools.partial(
        pl.kernel,
        mesh=mesh,
        out_type=jax.ShapeDtypeStruct((n, h), features.dtype),
        scratch_types=[
            pltpu.VMEM((n,), jnp.int32),
        ],
    )
    def scatter_rows(feat_hbm, q_hbm, out_hbm, q_v):
        wid = lax.axis_index("s") * _NUM_CORES + lax.axis_index("c")

        @pl.when(wid < n)
        def _():
            pltpu.sync_copy(q_hbm, q_v)
            off = jnp.minimum(wid, n - _LANES)
            vec = q_v[pl.ds(off, _LANES)]
            lane = wid - off
            qw = vec[0]
            for l in range(1, _LANES):
                qw = jnp.where(lane == l, vec[l], qw)
            pltpu.sync_copy(
                feat_hbm.at[pl.ds(wid, 1)], out_hbm.at[pl.ds(qw, 1)]
            )

    out = scatter_rows(features, query_letters.astype(jnp.int32))
    return out.reshape(-1)


# speedup vs baseline: 1.7473x; 1.7473x over previous
import functools

import jax
import jax.numpy as jnp
from jax import lax
from jax.experimental import pallas as pl
from jax.experimental.pallas import tpu as pltpu
from jax.experimental.pallas import tpu_sc as plsc

_NUM_CORES = 2
_LANES = 16
_SEG = 8


def kernel(features, query_letters):
    n, h = features.shape
    mesh = plsc.VectorSubcoreMesh(core_axis_name="c", subcore_axis_name="s")

    @functools.partial(
        pl.kernel,
        mesh=mesh,
        out_type=jax.ShapeDtypeStruct((n * h,), features.dtype),
        scratch_types=[
            pltpu.VMEM((_LANES,), jnp.int32),
            pltpu.VMEM((h,), features.dtype),
            pltpu.SemaphoreType.DMA,
        ],
    )
    def scatter_rows(feat_hbm, q_hbm, out_hbm, q_v, row_v, sem):
        wid = lax.axis_index("s") * _NUM_CORES + lax.axis_index("c")

        @pl.when(wid < n)
        def _():
            row_in = pltpu.make_async_copy(
                feat_hbm.at[pl.ds(wid * h, h)], row_v, sem
            )
            row_in.start()
            base = (wid // _SEG) * _SEG
            pltpu.sync_copy(q_hbm.at[pl.ds(base, _SEG)], q_v.at[pl.ds(0, _SEG)])
            vec = q_v[...]
            lane = wid - base
            qw = vec[0]
            for l in range(1, _SEG):
                qw = jnp.where(lane == l, vec[l], qw)
            row_in.wait()
            pltpu.sync_copy(row_v, out_hbm.at[pl.ds(qw * h, h)])

    out = scatter_rows(features.reshape(-1), query_letters.astype(jnp.int32))
    return out


# speedup vs baseline: 1.7640x; 1.0096x over previous
import functools

import jax
import jax.numpy as jnp
from jax import lax
from jax.experimental import pallas as pl
from jax.experimental.pallas import tpu as pltpu
from jax.experimental.pallas import tpu_sc as plsc

_NUM_CORES = 2
_LANES = 16
_SEG = 8


def kernel(features, query_letters):
    n, h = features.shape
    mesh = plsc.VectorSubcoreMesh(core_axis_name="c", subcore_axis_name="s")

    @functools.partial(
        pl.kernel,
        mesh=mesh,
        out_type=jax.ShapeDtypeStruct((n * h,), features.dtype),
        scratch_types=[
            pltpu.VMEM((_LANES,), jnp.int32),
            pltpu.VMEM((h,), features.dtype),
            pltpu.SemaphoreType.DMA,
        ],
    )
    def scatter_rows(feat_hbm, q_hbm, out_hbm, q_v, row_v, sem):
        wid = lax.axis_index("s") * _NUM_CORES + lax.axis_index("c")

        @pl.when(wid < n)
        def _():
            row_in = pltpu.make_async_copy(feat_hbm.at[wid], row_v, sem)
            row_in.start()
            base = (wid // _SEG) * _SEG
            pltpu.sync_copy(q_hbm.at[pl.ds(base, _SEG)], q_v.at[pl.ds(0, _SEG)])
            vec = q_v[...]
            lane = wid - base
            qw = vec[0]
            for l in range(1, _SEG):
                qw = jnp.where(lane == l, vec[l], qw)
            row_in.wait()
            pltpu.sync_copy(row_v, out_hbm.at[pl.ds(qw * h, h)])

    out = scatter_rows(features, query_letters.astype(jnp.int32))
    return out
